# Initial kernel scaffold; baseline (speedup 1.0000x reference)
import functools

import jax
import jax.numpy as jnp
from jax import lax
from jax.experimental import pallas as pl
from jax.experimental.pallas import tpu as pltpu
from jax.experimental.pallas import tpu_sc as plsc

NC = 2
NS = 16
NW = NC * NS
CH = 80
ZR = 125


def _make_agg(n, e, w):
    epw = e // NW
    n_it = epw // CH
    rps = n // NS
    n_z = rps // ZR
    mesh = plsc.VectorSubcoreMesh(core_axis_name="c", subcore_axis_name="s")

    @functools.partial(
        pl.kernel,
        mesh=mesh,
        out_type=jax.ShapeDtypeStruct((NC, n, w), jnp.float32),
        scratch_types=[
            pltpu.VMEM((CH,), jnp.int32),
            pltpu.VMEM((CH,), jnp.int32),
            pltpu.VMEM((CH, w), jnp.float32),
            pltpu.VMEM((ZR, w), jnp.float32),
            pltpu.VMEM_SHARED((n, w), jnp.float32),
            pltpu.SemaphoreType.DMA,
        ],
    )
    def agg(y_hbm, src_hbm, dst_hbm, zeros_hbm, out_hbm,
            sidx_v, didx_v, rows_v, zbuf_v, acc_sh, sem):
        cid = lax.axis_index("c")
        sid = lax.axis_index("s")
        wid = sid * NC + cid
        pltpu.sync_copy(zeros_hbm, zbuf_v)
        for k in range(n_z):
            pltpu.sync_copy(zbuf_v, acc_sh.at[pl.ds(sid * rps + k * ZR, ZR)])
        plsc.subcore_barrier()

        def body(i, _):
            base = wid * epw + i * CH
            pltpu.sync_copy(src_hbm.at[pl.ds(base, CH)], sidx_v)
            pltpu.sync_copy(dst_hbm.at[pl.ds(base, CH)], didx_v)
            pltpu.async_copy(y_hbm.at[sidx_v], rows_v, sem).wait()
            pltpu.sync_copy(rows_v, acc_sh.at[didx_v], add=True)
            return 0

        lax.fori_loop(0, n_it, body, 0)
        plsc.subcore_barrier()

        for k in range(n_z):
            off = sid * rps + k * ZR
            pltpu.sync_copy(acc_sh.at[pl.ds(off, ZR)], zbuf_v)
            pltpu.sync_copy(zbuf_v, out_hbm.at[cid, pl.ds(off, ZR)])

    return agg


def _make_edge_gather(n, e, w):
    epw = e // NW
    n_it = epw // CH
    mesh = plsc.VectorSubcoreMesh(core_axis_name="c", subcore_axis_name="s")

    @functools.partial(
        pl.kernel,
        mesh=mesh,
        out_type=[jax.ShapeDtypeStruct((e, w), jnp.float32),
                  jax.ShapeDtypeStruct((e, w), jnp.float32)],
        scratch_types=[
            pltpu.VMEM((CH,), jnp.int32),
            pltpu.VMEM((CH, w), jnp.float32),
            pltpu.SemaphoreType.DMA,
        ],
    )
    def gat(h_hbm, src_hbm, dst_hbm, hs_hbm, hd_hbm, idx_v, rows_v, sem):
        cid = lax.axis_index("c")
        sid = lax.axis_index("s")
        wid = sid * NC + cid

        def body(i, _):
            base = wid * epw + i * CH
            pltpu.sync_copy(src_hbm.at[pl.ds(base, CH)], idx_v)
            pltpu.async_copy(h_hbm.at[idx_v], rows_v, sem).wait()
            pltpu.sync_copy(rows_v, hs_hbm.at[pl.ds(base, CH)])
            pltpu.sync_copy(dst_hbm.at[pl.ds(base, CH)], idx_v)
            pltpu.async_copy(h_hbm.at[idx_v], rows_v, sem).wait()
            pltpu.sync_copy(rows_v, hd_hbm.at[pl.ds(base, CH)])
            return 0

        lax.fori_loop(0, n_it, body, 0)

    return gat


def _pre1_kernel(x_ref, wl_ref, wr_ref, y_ref, r_ref):
    xb = x_ref[...]
    y = jnp.dot(xb, wl_ref[...], preferred_element_type=jnp.float32)
    ones = jnp.ones((xb.shape[0], 16), jnp.float32)
    y_ref[...] = jnp.concatenate([y, ones], axis=1)
    r_ref[...] = jnp.dot(xb, wr_ref[...], preferred_element_type=jnp.float32)


def _post_mid_body(ps, cnt, r, bl, g, b, wl_ref, wr_ref, y_ref, r2_ref):
    mean = ps / jnp.maximum(cnt, 1.0)
    t = mean + bl + r
    nrm = jnp.sqrt(jnp.sum(t * t, axis=-1, keepdims=True))
    t = t / jnp.maximum(nrm, 1e-12)
    mu = jnp.mean(t, axis=-1, keepdims=True)
    var = jnp.mean((t - mu) ** 2, axis=-1, keepdims=True)
    h = jnp.maximum((t - mu) / jnp.sqrt(var + 1e-5) * g + b, 0.0)
    y_ref[...] = jnp.dot(h, wl_ref[...], preferred_element_type=jnp.float32)
    r2_ref[...] = jnp.dot(h, wr_ref[...], preferred_element_type=jnp.float32)


def _post1_kernel(p_ref, r_ref, bl_ref, g_ref, be_ref, wl_ref, wr_ref,
                  y_ref, r2_ref, cnt_ref):
    ps = p_ref[0] + p_ref[1]
    cnt = ps[:, 64:65]
    cnt_ref[...] = cnt
    _post_mid_body(ps[:, :64], cnt, r_ref[...], bl_ref[...], g_ref[...],
                   be_ref[...], wl_ref, wr_ref, y_ref, r2_ref)


def _post2_kernel(p_ref, r_ref, bl_ref, g_ref, be_ref, wl_ref, wr_ref,
                  cnt_ref, y_ref, r2_ref):
    ps = p_ref[0] + p_ref[1]
    _post_mid_body(ps, cnt_ref[...], r_ref[...], bl_ref[...], g_ref[...],
                   be_ref[...], wl_ref, wr_ref, y_ref, r2_ref)


def _post3_kernel(p_ref, r_ref, bl_ref, cnt_ref, h_ref):
    ps = p_ref[0] + p_ref[1]
    t = ps / jnp.maximum(cnt_ref[...], 1.0) + bl_ref[...] + r_ref[...]
    nrm = jnp.sqrt(jnp.sum(t * t, axis=-1, keepdims=True))
    h_ref[...] = t / jnp.maximum(nrm, 1e-12)


def _mlp_kernel(hs_ref, hd_ref, w1_ref, b1_ref, w2_ref, b2_ref, w3_ref,
                b3_ref, o_ref):
    hs = hs_ref[...]
    hd = hd_ref[...]
    w1 = w1_ref[...]
    z = (jnp.dot(hs, w1[0:32], preferred_element_type=jnp.float32)
         + jnp.dot(hd, w1[32:64], preferred_element_type=jnp.float32)
         + jnp.dot(hs * hd, w1[64:96], preferred_element_type=jnp.float32)
         + b1_ref[...])
    z = jnp.maximum(z, 0.0)
    z = jnp.maximum(jnp.dot(z, w2_ref[...], preferred_element_type=jnp.float32)
                    + b2_ref[...], 0.0)
    z = jnp.dot(z, w3_ref[...], preferred_element_type=jnp.float32) + b3_ref[...]
    o_ref[...] = jax.nn.sigmoid(z)


def _full(shape):
    return pl.BlockSpec(shape, lambda i: tuple(0 for _ in shape))


def _rows(bs, w):
    return pl.BlockSpec((bs, w), lambda i: (i, 0))


def kernel(x, edge_index, W1l, b1l, W1r, W2l, b2l, W2r, W3l, b3l, W3r,
           g1, be1, g2, be2, mW1, mb1, mW2, mb2, mW3, mb3):
    n, d_in = x.shape
    e = edge_index.shape[1]
    h_dim = W1l.shape[1]
    out_dim = W3l.shape[1]
    src = edge_index[0].astype(jnp.int32)
    dst = edge_index[1].astype(jnp.int32)

    bn = 1000
    gn = n // bn
    be_blk = 2560
    ge = e // be_blk

    y1, r1 = pl.pallas_call(
        _pre1_kernel,
        grid=(gn,),
        in_specs=[_rows(bn, d_in), _full((d_in, h_dim)), _full((d_in, h_dim))],
        out_specs=[_rows(bn, h_dim + 16), _rows(bn, h_dim)],
        out_shape=[jax.ShapeDtypeStruct((n, h_dim + 16), jnp.float32),
                   jax.ShapeDtypeStruct((n, h_dim), jnp.float32)],
    )(x, W1l, W1r)

    z80 = jnp.zeros((ZR, h_dim + 16), jnp.float32)
    p1 = _make_agg(n, e, h_dim + 16)(y1, src, dst, z80)

    y2, r2, cnt = pl.pallas_call(
        _post1_kernel,
        grid=(gn,),
        in_specs=[pl.BlockSpec((NC, bn, h_dim + 16), lambda i: (0, i, 0)),
                  _rows(bn, h_dim), _full((1, h_dim)), _full((1, h_dim)),
                  _full((1, h_dim)), _full((h_dim, h_dim)),
                  _full((h_dim, h_dim))],
        out_specs=[_rows(bn, h_dim), _rows(bn, h_dim), _rows(bn, 1)],
        out_shape=[jax.ShapeDtypeStruct((n, h_dim), jnp.float32),
                   jax.ShapeDtypeStruct((n, h_dim), jnp.float32),
                   jax.ShapeDtypeStruct((n, 1), jnp.float32)],
    )(p1, r1, b1l.reshape(1, -1), g1.reshape(1, -1), be1.reshape(1, -1),
      W2l, W2r)

    z64 = jnp.zeros((ZR, h_dim), jnp.float32)
    p2 = _make_agg(n, e, h_dim)(y2, src, dst, z64)
    y3, r3 = pl.pallas_call(
        _post2_kernel,
        grid=(gn,),
        in_specs=[pl.BlockSpec((NC, bn, h_dim), lambda i: (0, i, 0)),
                  _rows(bn, h_dim), _full((1, h_dim)), _full((1, h_dim)),
                  _full((1, h_dim)), _full((h_dim, out_dim)),
                  _full((h_dim, out_dim)), _rows(bn, 1)],
        out_specs=[_rows(bn, out_dim), _rows(bn, out_dim)],
        out_shape=[jax.ShapeDtypeStruct((n, out_dim), jnp.float32),
                   jax.ShapeDtypeStruct((n, out_dim), jnp.float32)],
    )(p2, r2, b2l.reshape(1, -1), g2.reshape(1, -1), be2.reshape(1, -1),
      W3l, W3r, cnt)

    z32 = jnp.zeros((ZR, out_dim), jnp.float32)
    p3 = _make_agg(n, e, out_dim)(y3, src, dst, z32)
    h3 = pl.pallas_call(
        _post3_kernel,
        grid=(gn,),
        in_specs=[pl.BlockSpec((NC, bn, out_dim), lambda i: (0, i, 0)),
                  _rows(bn, out_dim), _full((1, out_dim)), _rows(bn, 1)],
        out_specs=_rows(bn, out_dim),
        out_shape=jax.ShapeDtypeStruct((n, out_dim), jnp.float32),
    )(p3, r3, b3l.reshape(1, -1), cnt)

    hs, hd = _make_edge_gather(n, e, out_dim)(h3, src, dst)
    out = pl.pallas_call(
        _mlp_kernel,
        grid=(ge,),
        in_specs=[_rows(be_blk, out_dim), _rows(be_blk, out_dim),
                  _full((3 * out_dim, 64)), _full((1, 64)),
                  _full((64, 32)), _full((1, 32)),
                  _full((32, 1)), _full((1, 1))],
        out_specs=_rows(be_blk, 1),
        out_shape=jax.ShapeDtypeStruct((e, 1), jnp.float32),
    )(hs, hd, mW1, mb1.reshape(1, -1), mW2, mb2.reshape(1, -1),
      mW3, mb3.reshape(1, -1))
    return out[:, 0]


# speedup vs baseline: 3.3843x; 3.3843x over previous
import functools

import jax
import jax.numpy as jnp
from jax import lax
from jax.experimental import pallas as pl
from jax.experimental.pallas import tpu as pltpu
from jax.experimental.pallas import tpu_sc as plsc

NC = 2
NS = 16
NW = NC * NS
CH = 80
ZR = 128
WL = 128


def _make_agg(n, e):
    epw = e // NW
    n_it = epw // CH
    npad = -(-n // (NS * ZR)) * (NS * ZR)
    rps = npad // NS
    n_z = rps // ZR
    mesh = plsc.VectorSubcoreMesh(core_axis_name="c", subcore_axis_name="s")

    @functools.partial(
        pl.kernel,
        mesh=mesh,
        out_type=jax.ShapeDtypeStruct((NC, npad, WL), jnp.float32),
        scratch_types=[
            pltpu.VMEM((CH,), jnp.int32),
            pltpu.VMEM((CH,), jnp.int32),
            pltpu.VMEM((CH, WL), jnp.float32),
            pltpu.VMEM((ZR, WL), jnp.float32),
            pltpu.VMEM_SHARED((npad, WL), jnp.float32),
            pltpu.SemaphoreType.DMA,
        ],
    )
    def agg(y_hbm, src_hbm, dst_hbm, zeros_hbm, out_hbm,
            sidx_v, didx_v, rows_v, zbuf_v, acc_sh, sem):
        cid = lax.axis_index("c")
        sid = lax.axis_index("s")
        wid = sid * NC + cid
        pltpu.sync_copy(zeros_hbm, zbuf_v)
        for k in range(n_z):
            pltpu.sync_copy(zbuf_v, acc_sh.at[pl.ds(sid * rps + k * ZR, ZR)])
        plsc.subcore_barrier()

        def body(i, _):
            base = wid * epw + i * CH
            pltpu.sync_copy(src_hbm.at[pl.ds(base, CH)], sidx_v)
            pltpu.sync_copy(dst_hbm.at[pl.ds(base, CH)], didx_v)
            pltpu.async_copy(y_hbm.at[sidx_v], rows_v, sem).wait()
            pltpu.sync_copy(rows_v, acc_sh.at[didx_v], add=True)
            return 0

        lax.fori_loop(0, n_it, body, 0)
        plsc.subcore_barrier()

        for k in range(n_z):
            off = sid * rps + k * ZR
            pltpu.sync_copy(acc_sh.at[pl.ds(off, ZR)], zbuf_v)
            pltpu.sync_copy(zbuf_v, out_hbm.at[cid, pl.ds(off, ZR)])

    return agg


def _make_edge_gather(n, e):
    epw = e // NW
    n_it = epw // CH
    mesh = plsc.VectorSubcoreMesh(core_axis_name="c", subcore_axis_name="s")

    @functools.partial(
        pl.kernel,
        mesh=mesh,
        out_type=[jax.ShapeDtypeStruct((e, WL), jnp.float32),
                  jax.ShapeDtypeStruct((e, WL), jnp.float32)],
        scratch_types=[
            pltpu.VMEM((CH,), jnp.int32),
            pltpu.VMEM((CH, WL), jnp.float32),
            pltpu.SemaphoreType.DMA,
        ],
    )
    def gat(h_hbm, src_hbm, dst_hbm, hs_hbm, hd_hbm, idx_v, rows_v, sem):
        cid = lax.axis_index("c")
        sid = lax.axis_index("s")
        wid = sid * NC + cid

        def body(i, _):
            base = wid * epw + i * CH
            pltpu.sync_copy(src_hbm.at[pl.ds(base, CH)], idx_v)
            pltpu.async_copy(h_hbm.at[idx_v], rows_v, sem).wait()
            pltpu.sync_copy(rows_v, hs_hbm.at[pl.ds(base, CH)])
            pltpu.sync_copy(dst_hbm.at[pl.ds(base, CH)], idx_v)
            pltpu.async_copy(h_hbm.at[idx_v], rows_v, sem).wait()
            pltpu.sync_copy(rows_v, hd_hbm.at[pl.ds(base, CH)])
            return 0

        lax.fori_loop(0, n_it, body, 0)

    return gat


def _pad_cols(a, width):
    pad = width - a.shape[1]
    if pad == 0:
        return a
    return jnp.concatenate([a, jnp.zeros((a.shape[0], pad), jnp.float32)],
                           axis=1)


def _pre1_kernel(x_ref, wl_ref, wr_ref, y_ref, r_ref):
    xb = x_ref[...]
    y = jnp.dot(xb, wl_ref[...], preferred_element_type=jnp.float32)
    ones = jnp.ones((xb.shape[0], WL - y.shape[1]), jnp.float32)
    y_ref[...] = jnp.concatenate([y, ones], axis=1)
    r_ref[...] = jnp.dot(xb, wr_ref[...], preferred_element_type=jnp.float32)


def _post_mid_body(ps, cnt, r, bl, g, b, wl_ref, wr_ref, y_ref, r2_ref):
    mean = ps / jnp.maximum(cnt, 1.0)
    t = mean + bl + r
    nrm = jnp.sqrt(jnp.sum(t * t, axis=-1, keepdims=True))
    t = t / jnp.maximum(nrm, 1e-12)
    mu = jnp.mean(t, axis=-1, keepdims=True)
    var = jnp.mean((t - mu) ** 2, axis=-1, keepdims=True)
    h = jnp.maximum((t - mu) / jnp.sqrt(var + 1e-5) * g + b, 0.0)
    y = jnp.dot(h, wl_ref[...], preferred_element_type=jnp.float32)
    y_ref[...] = _pad_cols(y, WL)
    r2_ref[...] = jnp.dot(h, wr_ref[...], preferred_element_type=jnp.float32)


def _post1_kernel(p_ref, r_ref, bl_ref, g_ref, be_ref, wl_ref, wr_ref,
                  y_ref, r2_ref, cnt_ref):
    ps = p_ref[0] + p_ref[1]
    cnt = ps[:, 64:65]
    cnt_ref[...] = cnt
    _post_mid_body(ps[:, :64], cnt, r_ref[...], bl_ref[...], g_ref[...],
                   be_ref[...], wl_ref, wr_ref, y_ref, r2_ref)


def _post2_kernel(p_ref, r_ref, bl_ref, g_ref, be_ref, wl_ref, wr_ref,
                  cnt_ref, y_ref, r2_ref):
    ps = p_ref[0] + p_ref[1]
    _post_mid_body(ps[:, :64], cnt_ref[...], r_ref[...], bl_ref[...],
                   g_ref[...], be_ref[...], wl_ref, wr_ref, y_ref, r2_ref)


def _post3_kernel(p_ref, r_ref, bl_ref, cnt_ref, h_ref):
    ps = p_ref[0] + p_ref[1]
    t = ps[:, :32] / jnp.maximum(cnt_ref[...], 1.0) + bl_ref[...] + r_ref[...]
    nrm = jnp.sqrt(jnp.sum(t * t, axis=-1, keepdims=True))
    h_ref[...] = _pad_cols(t / jnp.maximum(nrm, 1e-12), WL)


def _mlp_kernel(hs_ref, hd_ref, w1_ref, b1_ref, w2_ref, b2_ref, w3_ref,
                b3_ref, o_ref):
    hs = hs_ref[:, 0:32]
    hd = hd_ref[:, 0:32]
    w1 = w1_ref[...]
    z = (jnp.dot(hs, w1[0:32], preferred_element_type=jnp.float32)
         + jnp.dot(hd, w1[32:64], preferred_element_type=jnp.float32)
         + jnp.dot(hs * hd, w1[64:96], preferred_element_type=jnp.float32)
         + b1_ref[...])
    z = jnp.maximum(z, 0.0)
    z = jnp.maximum(jnp.dot(z, w2_ref[...], preferred_element_type=jnp.float32)
                    + b2_ref[...], 0.0)
    z = jnp.dot(z, w3_ref[...], preferred_element_type=jnp.float32) + b3_ref[...]
    o_ref[...] = jax.nn.sigmoid(z)


def _full(shape):
    return pl.BlockSpec(shape, lambda i: tuple(0 for _ in shape))


def _rows(bs, w):
    return pl.BlockSpec((bs, w), lambda i: (i, 0))


def kernel(x, edge_index, W1l, b1l, W1r, W2l, b2l, W2r, W3l, b3l, W3r,
           g1, be1, g2, be2, mW1, mb1, mW2, mb2, mW3, mb3):
    n, d_in = x.shape
    e = edge_index.shape[1]
    h_dim = W1l.shape[1]
    out_dim = W3l.shape[1]
    src = edge_index[0].astype(jnp.int32)
    dst = edge_index[1].astype(jnp.int32)

    bn = 1000
    gn = n // bn
    be_blk = 2560
    ge = e // be_blk

    y1, r1 = pl.pallas_call(
        _pre1_kernel,
        grid=(gn,),
        in_specs=[_rows(bn, d_in), _full((d_in, h_dim)), _full((d_in, h_dim))],
        out_specs=[_rows(bn, WL), _rows(bn, h_dim)],
        out_shape=[jax.ShapeDtypeStruct((n, WL), jnp.float32),
                   jax.ShapeDtypeStruct((n, h_dim), jnp.float32)],
    )(x, W1l, W1r)

    zrs = jnp.zeros((ZR, WL), jnp.float32)
    agg = _make_agg(n, e)
    p1 = agg(y1, src, dst, zrs)

    y2, r2, cnt = pl.pallas_call(
        _post1_kernel,
        grid=(gn,),
        in_specs=[pl.BlockSpec((NC, bn, WL), lambda i: (0, i, 0)),
                  _rows(bn, h_dim), _full((1, h_dim)), _full((1, h_dim)),
                  _full((1, h_dim)), _full((h_dim, h_dim)),
                  _full((h_dim, h_dim))],
        out_specs=[_rows(bn, WL), _rows(bn, h_dim), _rows(bn, 1)],
        out_shape=[jax.ShapeDtypeStruct((n, WL), jnp.float32),
                   jax.ShapeDtypeStruct((n, h_dim), jnp.float32),
                   jax.ShapeDtypeStruct((n, 1), jnp.float32)],
    )(p1, r1, b1l.reshape(1, -1), g1.reshape(1, -1), be1.reshape(1, -1),
      W2l, W2r)

    p2 = agg(y2, src, dst, zrs)
    y3, r3 = pl.pallas_call(
        _post2_kernel,
        grid=(gn,),
        in_specs=[pl.BlockSpec((NC, bn, WL), lambda i: (0, i, 0)),
                  _rows(bn, h_dim), _full((1, h_dim)), _full((1, h_dim)),
                  _full((1, h_dim)), _full((h_dim, out_dim)),
                  _full((h_dim, out_dim)), _rows(bn, 1)],
        out_specs=[_rows(bn, WL), _rows(bn, out_dim)],
        out_shape=[jax.ShapeDtypeStruct((n, WL), jnp.float32),
                   jax.ShapeDtypeStruct((n, out_dim), jnp.float32)],
    )(p2, r2, b2l.reshape(1, -1), g2.reshape(1, -1), be2.reshape(1, -1),
      W3l, W3r, cnt)

    p3 = agg(y3, src, dst, zrs)
    h3 = pl.pallas_call(
        _post3_kernel,
        grid=(gn,),
        in_specs=[pl.BlockSpec((NC, bn, WL), lambda i: (0, i, 0)),
                  _rows(bn, out_dim), _full((1, out_dim)), _rows(bn, 1)],
        out_specs=_rows(bn, WL),
        out_shape=jax.ShapeDtypeStruct((n, WL), jnp.float32),
    )(p3, r3, b3l.reshape(1, -1), cnt)

    hs, hd = _make_edge_gather(n, e)(h3, src, dst)
    out = pl.pallas_call(
        _mlp_kernel,
        grid=(ge,),
        in_specs=[_rows(be_blk, WL), _rows(be_blk, WL),
                  _full((3 * out_dim, 64)), _full((1, 64)),
                  _full((64, 32)), _full((1, 32)),
                  _full((32, 1)), _full((1, 1))],
        out_specs=_rows(be_blk, 1),
        out_shape=jax.ShapeDtypeStruct((e, 1), jnp.float32),
    )(hs, hd, mW1, mb1.reshape(1, -1), mW2, mb2.reshape(1, -1),
      mW3, mb3.reshape(1, -1))
    return out[:, 0]


# speedup vs baseline: 6.8571x; 2.0261x over previous
import functools

import jax
import jax.numpy as jnp
from jax import lax
from jax.experimental import pallas as pl
from jax.experimental.pallas import tpu as pltpu
from jax.experimental.pallas import tpu_sc as plsc

NC = 2
NS = 16
NW = NC * NS
CH = 80
ZR = 128
WL = 128
RB = 5


def _make_agg(n, e):
    epw = e // NW
    n_ch = epw // CH
    rb = 3
    n_out = n_ch // rb
    rem = n_ch - n_out * rb
    npad = -(-n // (NS * ZR)) * (NS * ZR)
    rps = npad // NS
    mesh = plsc.VectorSubcoreMesh(core_axis_name="c", subcore_axis_name="s")

    @functools.partial(
        pl.kernel,
        mesh=mesh,
        out_type=jax.ShapeDtypeStruct((NC, npad, WL), jnp.float32),
        scratch_types=[
            pltpu.VMEM((n_ch, CH), jnp.int32),
            pltpu.VMEM((rb, CH), jnp.int32),
            pltpu.VMEM((rb, CH), jnp.int32),
            pltpu.VMEM((rb, CH, WL), jnp.float32),
            pltpu.VMEM_SHARED((npad, WL), jnp.float32),
            pltpu.SemaphoreType.DMA,
            pltpu.SemaphoreType.DMA,
            pltpu.SemaphoreType.DMA,
        ],
    )
    def agg(y_hbm, comb_hbm, zeros_hbm, out_hbm,
            comb_v, sidx_v, didx_v, rows_v, acc_sh, s0, s1, s2):
        sems = (s0, s1, s2)
        cid = lax.axis_index("c")
        sid = lax.axis_index("s")
        wid = sid * NC + cid

        def unpack_and_fire(g, b):
            for t in range(CH // 16):
                v = comb_v[g, pl.ds(t * 16, 16)]
                sidx_v[b, pl.ds(t * 16, 16)] = lax.bitwise_and(v, 0xFFFF)
                didx_v[b, pl.ds(t * 16, 16)] = lax.shift_right_logical(v, 16)
            pltpu.async_copy(y_hbm.at[sidx_v.at[b]], rows_v.at[b], sems[b])

        def drain_and_scatter(g, b):
            pltpu.make_async_copy(y_hbm.at[sidx_v.at[b]], rows_v.at[b],
                                  sems[b]).wait()
            pltpu.sync_copy(rows_v.at[b], acc_sh.at[didx_v.at[b]], add=True)

        pltpu.sync_copy(comb_hbm.at[wid], comb_v)
        pltpu.sync_copy(zeros_hbm, acc_sh.at[pl.ds(sid * rps, rps)])
        for b in range(rb):
            unpack_and_fire(b, b)
        plsc.subcore_barrier()

        def body(k, _):
            for b in range(rb):
                g = k * rb + b
                drain_and_scatter(g, b)

                @pl.when(g + rb < n_ch)
                def _():
                    unpack_and_fire(g + rb, b)
            return 0

        lax.fori_loop(0, n_out, body, 0)
        for j in range(rem):
            drain_and_scatter(n_out * rb + j, j)
        plsc.subcore_barrier()

        pltpu.sync_copy(acc_sh.at[pl.ds(sid * rps, rps)],
                        out_hbm.at[cid, pl.ds(sid * rps, rps)])

    return agg


def _make_edge_gather(n, e, w):
    ept = e // (NW // 2)
    n_ch = ept // CH
    n_out = n_ch // RB
    mesh = plsc.VectorSubcoreMesh(core_axis_name="c", subcore_axis_name="s")

    @functools.partial(
        pl.kernel,
        mesh=mesh,
        out_type=[jax.ShapeDtypeStruct((e, WL), jnp.float32),
                  jax.ShapeDtypeStruct((e, WL), jnp.float32)],
        scratch_types=[
            pltpu.VMEM((n_ch, CH), jnp.int32),
            pltpu.VMEM((RB, CH, WL), jnp.float32),
            pltpu.SemaphoreType.DMA,
            pltpu.SemaphoreType.DMA,
            pltpu.SemaphoreType.DMA,
            pltpu.SemaphoreType.DMA,
            pltpu.SemaphoreType.DMA,
        ],
    )
    def gat(h_hbm, src_hbm, dst_hbm, hs_hbm, hd_hbm,
            idx_v, rows_v, s0, s1, s2, s3, s4):
        sems = (s0, s1, s2, s3, s4)
        cid = lax.axis_index("c")
        sid = lax.axis_index("s")
        wid = sid * NC + cid
        dirn = wid % 2
        w2 = wid // 2

        def run(eidx_hbm, out_hbm):
            pltpu.sync_copy(eidx_hbm.at[w2], idx_v)
            for b in range(RB):
                pltpu.async_copy(h_hbm.at[idx_v.at[b]], rows_v.at[b], sems[b])

            def body(k, _):
                for b in range(RB):
                    g = k * RB + b
                    pltpu.make_async_copy(h_hbm.at[idx_v.at[g]], rows_v.at[b],
                                          sems[b]).wait()
                    base = (w2 * n_ch + g) * CH
                    pltpu.sync_copy(rows_v.at[b], out_hbm.at[pl.ds(base, CH)])

                    @pl.when(k < n_out - 1)
                    def _():
                        pltpu.async_copy(h_hbm.at[idx_v.at[g + RB]],
                                         rows_v.at[b], sems[b])
                return 0

            lax.fori_loop(0, n_out, body, 0)

        pl.when(dirn == 0)(lambda: run(src_hbm, hs_hbm))
        pl.when(dirn == 1)(lambda: run(dst_hbm, hd_hbm))

    return gat


def _pad_cols(a, width):
    pad = width - a.shape[1]
    if pad == 0:
        return a
    return jnp.concatenate([a, jnp.zeros((a.shape[0], pad), jnp.float32)],
                           axis=1)


def _pre1_kernel(x_ref, wl_ref, wr_ref, y_ref, r_ref):
    xb = x_ref[...]
    y = jnp.dot(xb, wl_ref[...], preferred_element_type=jnp.float32)
    ones = jnp.ones((xb.shape[0], WL - y.shape[1]), jnp.float32)
    y_ref[...] = jnp.concatenate([y, ones], axis=1)
    r_ref[...] = jnp.dot(xb, wr_ref[...], preferred_element_type=jnp.float32)


def _post_mid_body(ps, cnt, r, bl, g, b, wl_ref, wr_ref, y_ref, r2_ref):
    mean = ps / jnp.maximum(cnt, 1.0)
    t = mean + bl + r
    nrm = jnp.sqrt(jnp.sum(t * t, axis=-1, keepdims=True))
    t = t / jnp.maximum(nrm, 1e-12)
    mu = jnp.mean(t, axis=-1, keepdims=True)
    var = jnp.mean((t - mu) ** 2, axis=-1, keepdims=True)
    h = jnp.maximum((t - mu) / jnp.sqrt(var + 1e-5) * g + b, 0.0)
    y = jnp.dot(h, wl_ref[...], preferred_element_type=jnp.float32)
    y_ref[...] = _pad_cols(y, WL)
    r2_ref[...] = jnp.dot(h, wr_ref[...], preferred_element_type=jnp.float32)


def _post1_kernel(p_ref, r_ref, bl_ref, g_ref, be_ref, wl_ref, wr_ref,
                  y_ref, r2_ref, cnt_ref):
    ps = p_ref[0] + p_ref[1]
    cnt = ps[:, 64:65]
    cnt_ref[...] = cnt
    _post_mid_body(ps[:, :64], cnt, r_ref[...], bl_ref[...], g_ref[...],
                   be_ref[...], wl_ref, wr_ref, y_ref, r2_ref)


def _post2_kernel(p_ref, r_ref, bl_ref, g_ref, be_ref, wl_ref, wr_ref,
                  cnt_ref, y_ref, r2_ref):
    ps = p_ref[0] + p_ref[1]
    _post_mid_body(ps[:, :64], cnt_ref[...], r_ref[...], bl_ref[...],
                   g_ref[...], be_ref[...], wl_ref, wr_ref, y_ref, r2_ref)


def _post3_kernel(p_ref, r_ref, bl_ref, cnt_ref, h_ref):
    ps = p_ref[0] + p_ref[1]
    t = ps[:, :32] / jnp.maximum(cnt_ref[...], 1.0) + bl_ref[...] + r_ref[...]
    nrm = jnp.sqrt(jnp.sum(t * t, axis=-1, keepdims=True))
    h_ref[...] = _pad_cols(t / jnp.maximum(nrm, 1e-12), WL)


def _mlp_kernel(hs_ref, hd_ref, w1_ref, b1_ref, w2_ref, b2_ref, w3_ref,
                b3_ref, o_ref):
    hs = hs_ref[:, 0:32]
    hd = hd_ref[:, 0:32]
    w1 = w1_ref[...]
    z = (jnp.dot(hs, w1[0:32], preferred_element_type=jnp.float32)
         + jnp.dot(hd, w1[32:64], preferred_element_type=jnp.float32)
         + jnp.dot(hs * hd, w1[64:96], preferred_element_type=jnp.float32)
         + b1_ref[...])
    z = jnp.maximum(z, 0.0)
    z = jnp.maximum(jnp.dot(z, w2_ref[...], preferred_element_type=jnp.float32)
                    + b2_ref[...], 0.0)
    z = jnp.dot(z, w3_ref[...], preferred_element_type=jnp.float32) + b3_ref[...]
    o_ref[...] = jax.nn.sigmoid(z)


def _full(shape):
    return pl.BlockSpec(shape, lambda i: tuple(0 for _ in shape))


def _rows(bs, w):
    return pl.BlockSpec((bs, w), lambda i: (i, 0))


def kernel(x, edge_index, W1l, b1l, W1r, W2l, b2l, W2r, W3l, b3l, W3r,
           g1, be1, g2, be2, mW1, mb1, mW2, mb2, mW3, mb3):
    n, d_in = x.shape
    e = edge_index.shape[1]
    h_dim = W1l.shape[1]
    out_dim = W3l.shape[1]
    src = edge_index[0].astype(jnp.int32)
    dst = edge_index[1].astype(jnp.int32)
    epw = e // NW
    comb3 = (src | (dst << 16)).reshape(NW, epw // CH, CH)
    ept = e // (NW // 2)
    src3g = src.reshape(NW // 2, ept // CH, CH)
    dst3g = dst.reshape(NW // 2, ept // CH, CH)
    npad = -(-n // (NS * ZR)) * (NS * ZR)

    bn = 1000
    gn = n // bn
    be_blk = 2560
    ge = e // be_blk

    y1, r1 = pl.pallas_call(
        _pre1_kernel,
        grid=(gn,),
        in_specs=[_rows(bn, d_in), _full((d_in, h_dim)), _full((d_in, h_dim))],
        out_specs=[_rows(bn, WL), _rows(bn, h_dim)],
        out_shape=[jax.ShapeDtypeStruct((n, WL), jnp.float32),
                   jax.ShapeDtypeStruct((n, h_dim), jnp.float32)],
    )(x, W1l, W1r)

    zrs = jnp.zeros((npad // NS, WL), jnp.float32)
    agg = _make_agg(n, e)
    p1 = agg(y1, comb3, zrs)

    y2, r2, cnt = pl.pallas_call(
        _post1_kernel,
        grid=(gn,),
        in_specs=[pl.BlockSpec((NC, bn, WL), lambda i: (0, i, 0)),
                  _rows(bn, h_dim), _full((1, h_dim)), _full((1, h_dim)),
                  _full((1, h_dim)), _full((h_dim, h_dim)),
                  _full((h_dim, h_dim))],
        out_specs=[_rows(bn, WL), _rows(bn, h_dim), _rows(bn, 1)],
        out_shape=[jax.ShapeDtypeStruct((n, WL), jnp.float32),
                   jax.ShapeDtypeStruct((n, h_dim), jnp.float32),
                   jax.ShapeDtypeStruct((n, 1), jnp.float32)],
    )(p1, r1, b1l.reshape(1, -1), g1.reshape(1, -1), be1.reshape(1, -1),
      W2l, W2r)

    p2 = agg(y2, comb3, zrs)
    y3, r3 = pl.pallas_call(
        _post2_kernel,
        grid=(gn,),
        in_specs=[pl.BlockSpec((NC, bn, WL), lambda i: (0, i, 0)),
                  _rows(bn, h_dim), _full((1, h_dim)), _full((1, h_dim)),
                  _full((1, h_dim)), _full((h_dim, out_dim)),
                  _full((h_dim, out_dim)), _rows(bn, 1)],
        out_specs=[_rows(bn, WL), _rows(bn, out_dim)],
        out_shape=[jax.ShapeDtypeStruct((n, WL), jnp.float32),
                   jax.ShapeDtypeStruct((n, out_dim), jnp.float32)],
    )(p2, r2, b2l.reshape(1, -1), g2.reshape(1, -1), be2.reshape(1, -1),
      W3l, W3r, cnt)

    p3 = agg(y3, comb3, zrs)
    h3 = pl.pallas_call(
        _post3_kernel,
        grid=(gn,),
        in_specs=[pl.BlockSpec((NC, bn, WL), lambda i: (0, i, 0)),
                  _rows(bn, out_dim), _full((1, out_dim)), _rows(bn, 1)],
        out_specs=_rows(bn, WL),
        out_shape=jax.ShapeDtypeStruct((n, WL), jnp.float32),
    )(p3, r3, b3l.reshape(1, -1), cnt)

    hs, hd = _make_edge_gather(n, e, out_dim)(h3, src3g, dst3g)
    out = pl.pallas_call(
        _mlp_kernel,
        grid=(ge,),
        in_specs=[_rows(be_blk, WL), _rows(be_blk, WL),
                  _full((3 * out_dim, 64)), _full((1, 64)),
                  _full((64, 32)), _full((1, 32)),
                  _full((32, 1)), _full((1, 1))],
        out_specs=_rows(be_blk, 1),
        out_shape=jax.ShapeDtypeStruct((e, 1), jnp.float32),
    )(hs, hd, mW1, mb1.reshape(1, -1), mW2, mb2.reshape(1, -1),
      mW3, mb3.reshape(1, -1))
    return out[:, 0]


# speedup vs baseline: 7.3980x; 1.0789x over previous
import functools

import jax
import jax.numpy as jnp
from jax import lax
from jax.experimental import pallas as pl
from jax.experimental.pallas import tpu as pltpu
from jax.experimental.pallas import tpu_sc as plsc

NC = 2
NS = 16
NW = NC * NS
CH = 80
ZR = 128
WL = 128
RB = 5


def _make_agg(n, e):
    epw = e // NW
    n_ch = epw // CH
    rb = 3
    n_out = n_ch // rb
    rem = n_ch - n_out * rb
    npad = -(-n // (NS * ZR)) * (NS * ZR)
    rps = npad // NS
    mesh = plsc.VectorSubcoreMesh(core_axis_name="c", subcore_axis_name="s")

    @functools.partial(
        pl.kernel,
        mesh=mesh,
        out_type=jax.ShapeDtypeStruct((NC, npad, WL), jnp.float32),
        scratch_types=[
            pltpu.VMEM((n_ch, CH), jnp.int32),
            pltpu.VMEM((rb, CH), jnp.int32),
            pltpu.VMEM((rb, CH), jnp.int32),
            pltpu.VMEM((rb, CH, WL), jnp.float32),
            pltpu.VMEM_SHARED((npad, WL), jnp.float32),
            pltpu.SemaphoreType.DMA,
            pltpu.SemaphoreType.DMA,
            pltpu.SemaphoreType.DMA,
        ],
    )
    def agg(y_hbm, comb_hbm, zeros_hbm, out_hbm,
            comb_v, sidx_v, didx_v, rows_v, acc_sh, s0, s1, s2):
        sems = (s0, s1, s2)
        cid = lax.axis_index("c")
        sid = lax.axis_index("s")
        wid = sid * NC + cid

        def unpack_and_fire(g, b):
            for t in range(CH // 16):
                v = comb_v[g, pl.ds(t * 16, 16)]
                sidx_v[b, pl.ds(t * 16, 16)] = lax.bitwise_and(v, 0xFFFF)
                didx_v[b, pl.ds(t * 16, 16)] = lax.shift_right_logical(v, 16)
            pltpu.async_copy(y_hbm.at[sidx_v.at[b]], rows_v.at[b], sems[b])

        def drain_and_scatter(g, b):
            pltpu.make_async_copy(y_hbm.at[sidx_v.at[b]], rows_v.at[b],
                                  sems[b]).wait()
            pltpu.sync_copy(rows_v.at[b], acc_sh.at[didx_v.at[b]], add=True)

        pltpu.sync_copy(comb_hbm.at[wid], comb_v)
        pltpu.sync_copy(zeros_hbm, acc_sh.at[pl.ds(sid * rps, rps)])
        for b in range(rb):
            unpack_and_fire(b, b)
        plsc.subcore_barrier()

        def body(k, _):
            for b in range(rb):
                g = k * rb + b
                drain_and_scatter(g, b)

                @pl.when(g + rb < n_ch)
                def _():
                    unpack_and_fire(g + rb, b)
            return 0

        lax.fori_loop(0, n_out, body, 0)
        for j in range(rem):
            drain_and_scatter(n_out * rb + j, j)
        plsc.subcore_barrier()

        pltpu.sync_copy(acc_sh.at[pl.ds(sid * rps, rps)],
                        out_hbm.at[cid, pl.ds(sid * rps, rps)])

    return agg


def _make_edge_gather(n, e, w):
    epw = e // NW
    n_ch = epw // CH
    rb = 4
    n_out = n_ch // rb
    rem = n_ch - n_out * rb
    pkr = CH // 2
    mesh = plsc.VectorSubcoreMesh(core_axis_name="c", subcore_axis_name="s")

    @functools.partial(
        pl.kernel,
        mesh=mesh,
        out_type=jax.ShapeDtypeStruct((e // 2, WL), jnp.float32),
        scratch_types=[
            pltpu.VMEM((n_ch, CH), jnp.int32),
            pltpu.VMEM((rb, CH), jnp.int32),
            pltpu.VMEM((rb, CH), jnp.int32),
            pltpu.VMEM((rb, CH, WL), jnp.float32),
            pltpu.VMEM((rb, CH, WL), jnp.float32),
            pltpu.VMEM((rb, pkr, WL), jnp.float32),
            pltpu.SemaphoreType.DMA,
            pltpu.SemaphoreType.DMA,
            pltpu.SemaphoreType.DMA,
            pltpu.SemaphoreType.DMA,
            pltpu.SemaphoreType.DMA,
            pltpu.SemaphoreType.DMA,
            pltpu.SemaphoreType.DMA,
            pltpu.SemaphoreType.DMA,
        ],
    )
    def gat(h_hbm, comb_hbm, out_hbm, comb_v, sidx_v, didx_v,
            srows_v, drows_v, pk_v, g0, g1, g2, g3, w0, w1, w2, w3):
        gsem = (g0, g1, g2, g3)
        wsem = (w0, w1, w2, w3)
        cid = lax.axis_index("c")
        sid = lax.axis_index("s")
        wid = sid * NC + cid

        def unpack_and_fire(g, b):
            for t in range(CH // 16):
                v = comb_v[g, pl.ds(t * 16, 16)]
                sidx_v[b, pl.ds(t * 16, 16)] = lax.bitwise_and(v, 0xFFFF)
                didx_v[b, pl.ds(t * 16, 16)] = lax.shift_right_logical(v, 16)
            pltpu.async_copy(h_hbm.at[sidx_v.at[b]], srows_v.at[b], gsem[b])
            pltpu.async_copy(h_hbm.at[didx_v.at[b]], drows_v.at[b], gsem[b])

        def drain_pack_write(g, b, wait_prev):
            pltpu.make_async_copy(h_hbm.at[sidx_v.at[b]], srows_v.at[b],
                                  gsem[b]).wait()
            pltpu.make_async_copy(h_hbm.at[didx_v.at[b]], drows_v.at[b],
                                  gsem[b]).wait()

            @pl.when(wait_prev)
            def _():
                pltpu.make_async_copy(
                    pk_v.at[b], out_hbm.at[pl.ds(0, pkr)], wsem[b]).wait()

            for j in range(pkr):
                for t in range(w // 16):
                    pk_v[b, j, pl.ds(t * 16, 16)] = (
                        srows_v[b, 2 * j, pl.ds(t * 16, 16)])
                    pk_v[b, j, pl.ds(w + t * 16, 16)] = (
                        drows_v[b, 2 * j, pl.ds(t * 16, 16)])
                    pk_v[b, j, pl.ds(2 * w + t * 16, 16)] = (
                        srows_v[b, 2 * j + 1, pl.ds(t * 16, 16)])
                    pk_v[b, j, pl.ds(3 * w + t * 16, 16)] = (
                        drows_v[b, 2 * j + 1, pl.ds(t * 16, 16)])
            base = (wid * n_ch + g) * pkr
            pltpu.async_copy(pk_v.at[b], out_hbm.at[pl.ds(base, pkr)], wsem[b])

        pltpu.sync_copy(comb_hbm.at[wid], comb_v)
        for b in range(rb):
            unpack_and_fire(b, b)

        def body(k, _):
            for b in range(rb):
                g = k * rb + b
                drain_pack_write(g, b, k > 0)

                @pl.when(g + rb < n_ch)
                def _():
                    unpack_and_fire(g + rb, b)
            return 0

        lax.fori_loop(0, n_out, body, 0)
        for j in range(rem):
            drain_pack_write(n_out * rb + j, j, jnp.bool_(True))
        for b in range(rb):
            pltpu.make_async_copy(pk_v.at[b], out_hbm.at[pl.ds(0, pkr)],
                                  wsem[b]).wait()

    return gat


def _pad_cols(a, width):
    pad = width - a.shape[1]
    if pad == 0:
        return a
    return jnp.concatenate([a, jnp.zeros((a.shape[0], pad), jnp.float32)],
                           axis=1)


def _pre1_kernel(x_ref, wl_ref, wr_ref, y_ref, r_ref):
    xb = x_ref[...]
    y = jnp.dot(xb, wl_ref[...], preferred_element_type=jnp.float32)
    ones = jnp.ones((xb.shape[0], WL - y.shape[1]), jnp.float32)
    y_ref[...] = jnp.concatenate([y, ones], axis=1)
    r_ref[...] = jnp.dot(xb, wr_ref[...], preferred_element_type=jnp.float32)


def _post_mid_body(ps, cnt, r, bl, g, b, wl_ref, wr_ref, y_ref, r2_ref):
    mean = ps / jnp.maximum(cnt, 1.0)
    t = mean + bl + r
    nrm = jnp.sqrt(jnp.sum(t * t, axis=-1, keepdims=True))
    t = t / jnp.maximum(nrm, 1e-12)
    mu = jnp.mean(t, axis=-1, keepdims=True)
    var = jnp.mean((t - mu) ** 2, axis=-1, keepdims=True)
    h = jnp.maximum((t - mu) / jnp.sqrt(var + 1e-5) * g + b, 0.0)
    y = jnp.dot(h, wl_ref[...], preferred_element_type=jnp.float32)
    y_ref[...] = _pad_cols(y, WL)
    r2_ref[...] = jnp.dot(h, wr_ref[...], preferred_element_type=jnp.float32)


def _post1_kernel(p_ref, r_ref, bl_ref, g_ref, be_ref, wl_ref, wr_ref,
                  y_ref, r2_ref, cnt_ref):
    ps = p_ref[0] + p_ref[1]
    cnt = ps[:, 64:65]
    cnt_ref[...] = cnt
    _post_mid_body(ps[:, :64], cnt, r_ref[...], bl_ref[...], g_ref[...],
                   be_ref[...], wl_ref, wr_ref, y_ref, r2_ref)


def _post2_kernel(p_ref, r_ref, bl_ref, g_ref, be_ref, wl_ref, wr_ref,
                  cnt_ref, y_ref, r2_ref):
    ps = p_ref[0] + p_ref[1]
    _post_mid_body(ps[:, :64], cnt_ref[...], r_ref[...], bl_ref[...],
                   g_ref[...], be_ref[...], wl_ref, wr_ref, y_ref, r2_ref)


def _post3_kernel(p_ref, r_ref, bl_ref, cnt_ref, h_ref):
    ps = p_ref[0] + p_ref[1]
    t = ps[:, :32] / jnp.maximum(cnt_ref[...], 1.0) + bl_ref[...] + r_ref[...]
    nrm = jnp.sqrt(jnp.sum(t * t, axis=-1, keepdims=True))
    h_ref[...] = _pad_cols(t / jnp.maximum(nrm, 1e-12), WL)


def _mlp_kernel(pk_ref, w1_ref, b1_ref, w2_ref, b2_ref, w3_ref,
                b3_ref, o_ref):
    w1 = w1_ref[...]
    outs = []
    for g in range(2):
        hs = pk_ref[:, 64 * g:64 * g + 32]
        hd = pk_ref[:, 64 * g + 32:64 * g + 64]
        z = (jnp.dot(hs, w1[0:32], preferred_element_type=jnp.float32)
             + jnp.dot(hd, w1[32:64], preferred_element_type=jnp.float32)
             + jnp.dot(hs * hd, w1[64:96], preferred_element_type=jnp.float32)
             + b1_ref[...])
        z = jnp.maximum(z, 0.0)
        z = jnp.maximum(
            jnp.dot(z, w2_ref[...], preferred_element_type=jnp.float32)
            + b2_ref[...], 0.0)
        z = (jnp.dot(z, w3_ref[...], preferred_element_type=jnp.float32)
             + b3_ref[...])
        outs.append(jax.nn.sigmoid(z))
    o_ref[...] = jnp.concatenate(outs, axis=1)


def _full(shape):
    return pl.BlockSpec(shape, lambda i: tuple(0 for _ in shape))


def _rows(bs, w):
    return pl.BlockSpec((bs, w), lambda i: (i, 0))


def kernel(x, edge_index, W1l, b1l, W1r, W2l, b2l, W2r, W3l, b3l, W3r,
           g1, be1, g2, be2, mW1, mb1, mW2, mb2, mW3, mb3):
    n, d_in = x.shape
    e = edge_index.shape[1]
    h_dim = W1l.shape[1]
    out_dim = W3l.shape[1]
    src = edge_index[0].astype(jnp.int32)
    dst = edge_index[1].astype(jnp.int32)
    epw = e // NW
    comb3 = (src | (dst << 16)).reshape(NW, epw // CH, CH)
    npad = -(-n // (NS * ZR)) * (NS * ZR)

    bn = 1000
    gn = n // bn
    be_blk = 2560
    ge = e // be_blk

    y1, r1 = pl.pallas_call(
        _pre1_kernel,
        grid=(gn,),
        in_specs=[_rows(bn, d_in), _full((d_in, h_dim)), _full((d_in, h_dim))],
        out_specs=[_rows(bn, WL), _rows(bn, h_dim)],
        out_shape=[jax.ShapeDtypeStruct((n, WL), jnp.float32),
                   jax.ShapeDtypeStruct((n, h_dim), jnp.float32)],
    )(x, W1l, W1r)

    zrs = jnp.zeros((npad // NS, WL), jnp.float32)
    agg = _make_agg(n, e)
    p1 = agg(y1, comb3, zrs)

    y2, r2, cnt = pl.pallas_call(
        _post1_kernel,
        grid=(gn,),
        in_specs=[pl.BlockSpec((NC, bn, WL), lambda i: (0, i, 0)),
                  _rows(bn, h_dim), _full((1, h_dim)), _full((1, h_dim)),
                  _full((1, h_dim)), _full((h_dim, h_dim)),
                  _full((h_dim, h_dim))],
        out_specs=[_rows(bn, WL), _rows(bn, h_dim), _rows(bn, 1)],
        out_shape=[jax.ShapeDtypeStruct((n, WL), jnp.float32),
                   jax.ShapeDtypeStruct((n, h_dim), jnp.float32),
                   jax.ShapeDtypeStruct((n, 1), jnp.float32)],
    )(p1, r1, b1l.reshape(1, -1), g1.reshape(1, -1), be1.reshape(1, -1),
      W2l, W2r)

    p2 = agg(y2, comb3, zrs)
    y3, r3 = pl.pallas_call(
        _post2_kernel,
        grid=(gn,),
        in_specs=[pl.BlockSpec((NC, bn, WL), lambda i: (0, i, 0)),
                  _rows(bn, h_dim), _full((1, h_dim)), _full((1, h_dim)),
                  _full((1, h_dim)), _full((h_dim, out_dim)),
                  _full((h_dim, out_dim)), _rows(bn, 1)],
        out_specs=[_rows(bn, WL), _rows(bn, out_dim)],
        out_shape=[jax.ShapeDtypeStruct((n, WL), jnp.float32),
                   jax.ShapeDtypeStruct((n, out_dim), jnp.float32)],
    )(p2, r2, b2l.reshape(1, -1), g2.reshape(1, -1), be2.reshape(1, -1),
      W3l, W3r, cnt)

    p3 = agg(y3, comb3, zrs)
    h3 = pl.pallas_call(
        _post3_kernel,
        grid=(gn,),
        in_specs=[pl.BlockSpec((NC, bn, WL), lambda i: (0, i, 0)),
                  _rows(bn, out_dim), _full((1, out_dim)), _rows(bn, 1)],
        out_specs=_rows(bn, WL),
        out_shape=jax.ShapeDtypeStruct((n, WL), jnp.float32),
    )(p3, r3, b3l.reshape(1, -1), cnt)

    hsd = _make_edge_gather(n, e, out_dim)(h3, comb3)
    out = pl.pallas_call(
        _mlp_kernel,
        grid=(ge,),
        in_specs=[_rows(be_blk // 2, WL),
                  _full((3 * out_dim, 64)), _full((1, 64)),
                  _full((64, 32)), _full((1, 32)),
                  _full((32, 1)), _full((1, 1))],
        out_specs=_rows(be_blk // 2, 2),
        out_shape=jax.ShapeDtypeStruct((e // 2, 2), jnp.float32),
    )(hsd, mW1, mb1.reshape(1, -1), mW2, mb2.reshape(1, -1),
      mW3, mb3.reshape(1, -1))
    return out.reshape(e)
